# baseline (speedup 1.0000x reference)
import functools

import jax
import jax.numpy as jnp
from jax import lax
from jax.experimental import pallas as pl
from jax.experimental.pallas import tpu as pltpu
from jax.experimental.pallas import tpu_sc as plsc

_VOCAB = 1000
_LEN = 50
_DPOS = 64
_DOUT = _VOCAB + _DPOS
_NC = 2
_NS = 16
_NW = _NC * _NS
_LANES = 16
_XPAD = 64


def _sc_body(x_hbm, tmpl_hbm, out_hbm, x_v, tmpl_v, sems):
    wid = lax.axis_index("s") * _NC + lax.axis_index("c")
    bpw = out_hbm.shape[0] // _NW
    base = wid * bpw

    pltpu.sync_copy(x_hbm.at[pl.ds(base, bpw)], x_v)
    pltpu.sync_copy(tmpl_hbm, tmpl_v.at[0])
    pltpu.sync_copy(tmpl_hbm, tmpl_v.at[1])

    lane = lax.broadcasted_iota(jnp.int32, (_LANES,), 0)
    ones = jnp.full((_LANES,), 1.0, jnp.float32)
    zeros = jnp.zeros((_LANES,), jnp.float32)

    def scatter_row(buf, i, val):
        for c in range(_XPAD // _LANES):
            rows = lane + (c * _LANES)
            mask = rows < _LEN
            xs = x_v[i, pl.ds(c * _LANES, _LANES)]
            plsc.store_scatter(buf, [rows, xs], val, mask=mask)

    def issue(i):
        p = lax.rem(i, 2)
        scatter_row(tmpl_v.at[p], i, ones)
        pltpu.async_copy(tmpl_v.at[p], out_hbm.at[base + i], sems.at[p])

    def wait_and_clear(i):
        p = lax.rem(i, 2)
        pltpu.make_async_copy(tmpl_v.at[p], out_hbm.at[base + i],
                              sems.at[p]).wait()
        scatter_row(tmpl_v.at[p], i, zeros)

    issue(0)
    def step(i, carry):
        issue(i)
        wait_and_clear(i - 1)
        return carry
    lax.fori_loop(1, bpw, step, 0)
    wait_and_clear(bpw - 1)


def kernel(X, pos_emb):
    batch = X.shape[0]
    x64 = jnp.pad(X.astype(jnp.int32), ((0, 0), (0, _XPAD - _LEN)))
    tmpl = jnp.concatenate(
        [jnp.zeros((_LEN, _VOCAB), jnp.float32),
         pos_emb.astype(jnp.float32)], axis=1)
    run = pl.kernel(
        _sc_body,
        out_type=jax.ShapeDtypeStruct((batch, _LEN, _DOUT), jnp.float32),
        mesh=plsc.VectorSubcoreMesh(core_axis_name="c", subcore_axis_name="s",
                                    num_cores=_NC, num_subcores=_NS),
        compiler_params=pltpu.CompilerParams(use_tc_tiling_on_sc=False,
                                             needs_layout_passes=False),
        scratch_types=[
            pltpu.VMEM((batch // _NW, _XPAD), jnp.int32),
            pltpu.VMEM((2, _LEN, _DOUT), jnp.float32),
            pltpu.SemaphoreType.DMA((2,)),
        ],
    )
    return run(x64, tmpl)


# speedup vs baseline: 2.0110x; 2.0110x over previous
import functools

import jax
import jax.numpy as jnp
from jax import lax
from jax.experimental import pallas as pl
from jax.experimental.pallas import tpu as pltpu
from jax.experimental.pallas import tpu_sc as plsc

_VOCAB = 1000
_LEN = 50
_DPOS = 64
_DOUT = _VOCAB + _DPOS
_NC = 2
_NS = 16
_NW = _NC * _NS
_LANES = 16
_XPAD = 64


def _sc_body(x_hbm, tmpl_hbm, out_hbm, x_v, tmpl_v):
    wid = lax.axis_index("s") * _NC + lax.axis_index("c")
    bpw = out_hbm.shape[0] // _NW
    base = wid * bpw

    pltpu.sync_copy(x_hbm.at[pl.ds(base, bpw)], x_v)
    pltpu.sync_copy(tmpl_hbm, tmpl_v)

    lane = lax.broadcasted_iota(jnp.int32, (_LANES,), 0)
    ones = jnp.full((_LANES,), 1.0, jnp.float32)
    zeros = jnp.zeros((_LANES,), jnp.float32)

    def scatter_row(i, val):
        for c in range(_XPAD // _LANES):
            rows = lane + (c * _LANES)
            mask = rows < _LEN
            xs = x_v[i, pl.ds(c * _LANES, _LANES)]
            plsc.store_scatter(tmpl_v, [rows, xs], val, mask=mask)

    def step(i, carry):
        scatter_row(i, ones)
        pltpu.sync_copy(tmpl_v, out_hbm.at[base + i])
        scatter_row(i, zeros)
        return carry
    lax.fori_loop(0, bpw, step, 0)


def kernel(X, pos_emb):
    batch = X.shape[0]
    x64 = jnp.pad(X.astype(jnp.int32), ((0, 0), (0, _XPAD - _LEN)))
    tmpl = jnp.concatenate(
        [jnp.zeros((_LEN, _VOCAB), jnp.float32),
         pos_emb.astype(jnp.float32)], axis=1)
    run = pl.kernel(
        _sc_body,
        out_type=jax.ShapeDtypeStruct((batch, _LEN, _DOUT), jnp.float32),
        mesh=plsc.VectorSubcoreMesh(core_axis_name="c", subcore_axis_name="s",
                                    num_cores=_NC, num_subcores=_NS),
        compiler_params=pltpu.CompilerParams(needs_layout_passes=False),
        scratch_types=[
            pltpu.VMEM((batch // _NW, _XPAD), jnp.int32),
            pltpu.VMEM((_LEN, _DOUT), jnp.float32),
        ],
    )
    return run(x64, tmpl)
